# baseline (speedup 1.0000x reference)
import functools
import math

import jax
import jax.numpy as jnp
from jax import lax
from jax.experimental import pallas as pl
from jax.experimental.pallas import tpu as pltpu
from jax.experimental.pallas import tpu_sc as plsc

F32 = jnp.float32
I32 = jnp.int32


def _ru(x, m):
    return (x + m - 1) // m * m


def _ln_math(x, g, b):
    m = jnp.mean(x, -1, keepdims=True)
    v = jnp.mean((x - m) ** 2, -1, keepdims=True)
    return (x - m) / jnp.sqrt(v + 1e-5) * g + b


def _mish(x):
    return x * jnp.tanh(jax.nn.softplus(x))


def _prep_src_krn(ns, bn, s, r, x_ref, wg1_ref, wg2_ref, bg_ref, wes_ref,
                  g_ref, gmax_ref):
    i = pl.program_id(0)
    x = x_ref[...]
    u = jnp.dot(x, wg1_ref[...], preferred_element_type=F32)
    wneg = -(jnp.dot(x, wg2_ref[...], preferred_element_type=F32) + bg_ref[...])
    a = jnp.dot(x, wes_ref[...], preferred_element_type=F32)
    g_ref[:, 0:s] = x
    g_ref[:, s:2 * s] = u
    g_ref[:, 2 * s:3 * s] = wneg
    g_ref[:, 3 * s:r] = jnp.concatenate(
        [a, jnp.zeros((bn, r - 3 * s - 1), F32)], axis=1)
    rid = lax.broadcasted_iota(I32, (bn, 1), 0) + i * bn
    xm = jnp.where(rid < ns, x, -1e30)
    bmax = jnp.max(xm, axis=0, keepdims=True)

    @pl.when(i == 0)
    def _():
        gmax_ref[...] = jnp.zeros_like(gmax_ref)

    gmax_ref[...] = jnp.maximum(gmax_ref[...], bmax)


def _prep_src(x, wg, bg, wes):
    ns, s = x.shape
    r = _ru(3 * s + 1, 128)
    bn = 512
    grid = (pl.cdiv(ns, bn),)
    krn = functools.partial(_prep_src_krn, ns, bn, s, r)
    return pl.pallas_call(
        krn,
        grid=grid,
        in_specs=[
            pl.BlockSpec((bn, s), lambda i: (i, 0)),
            pl.BlockSpec((s, s), lambda i: (0, 0)),
            pl.BlockSpec((s, s), lambda i: (0, 0)),
            pl.BlockSpec((s,), lambda i: (0,)),
            pl.BlockSpec((s, 1), lambda i: (0, 0)),
        ],
        out_specs=[
            pl.BlockSpec((bn, r), lambda i: (i, 0)),
            pl.BlockSpec((1, s), lambda i: (0, 0)),
        ],
        out_shape=[
            jax.ShapeDtypeStruct((ns, r), F32),
            jax.ShapeDtypeStruct((1, s), F32),
        ],
    )(x, wg[:s], wg[s:], bg, wes)


def _adst_krn(x_ref, w_ref, be_ref, o_ref):
    o_ref[...] = jnp.dot(x_ref[...], w_ref[...],
                         preferred_element_type=F32) + be_ref[...]


def _prep_dst(x, wed, be):
    nd, t = x.shape
    bn = min(512, _ru(nd, 8))
    grid = (pl.cdiv(nd, bn),)
    out = pl.pallas_call(
        _adst_krn,
        grid=grid,
        in_specs=[
            pl.BlockSpec((bn, t), lambda i: (i, 0)),
            pl.BlockSpec((t, 1), lambda i: (0, 0)),
            pl.BlockSpec((1,), lambda i: (0,)),
        ],
        out_specs=pl.BlockSpec((bn, 1), lambda i: (i, 0)),
        out_shape=jax.ShapeDtypeStruct((nd, 1), F32),
    )(x, wed, be)
    return out.reshape(-1)


def _make_edge_pass(ns, nd, s, nch, own_n, eb, nst):
    s2 = 2 * s
    r = _ru(3 * s + 1, 128)
    acol = 3 * s
    nf = s // 16
    b = 64
    dummy = own_n
    crow = 32 * own_n
    mesh = plsc.VectorSubcoreMesh(core_axis_name="c", subcore_axis_name="s")

    @functools.partial(
        pl.kernel,
        mesh=mesh,
        compiler_params=pltpu.CompilerParams(needs_layout_passes=False),
        out_type=jax.ShapeDtypeStruct((nch * crow, s2), F32),
        scratch_types=[
            pltpu.VMEM((2 * eb,), I32),
            pltpu.VMEM((2 * eb,), I32),
            pltpu.VMEM((b + 32,), I32),
            pltpu.VMEM((b + 32,), I32),
            pltpu.VMEM((b,), I32),
            pltpu.VMEM((b,), I32),
            pltpu.VMEM((b, r), F32),
            pltpu.VMEM((own_n + 16,), F32),
            pltpu.VMEM((16,), F32),
            pltpu.VMEM((16,), F32),
            pltpu.VMEM((s,), F32),
            pltpu.VMEM((own_n + 1, s2), F32),
            pltpu.SMEM((4,), I32),
            pltpu.SemaphoreType.DMA,
            pltpu.SemaphoreType.DMA,
            pltpu.SemaphoreType.DMA,
        ],
    )
    def krn(g_hbm, adst_hbm, gmax_hbm, src_hbm, dst_hbm, out_hbm,
            esrc, edst, psrc, ploc, psx, plx, rows, adst_v, ewb, cmb,
            gmax_v, acc, smem, sem, sems, semd):
        cid = lax.axis_index("c")
        sid = lax.axis_index("s")
        sid2 = cid * 16 + sid
        iota = lax.iota(I32, 16)
        zi = jnp.zeros((16,), I32)
        zf = jnp.zeros((16,), F32)

        for j in range(0, b + 32, 16):
            psrc[pl.ds(j, 16)] = zi
            ploc[pl.ds(j, 16)] = jnp.full((16,), dummy, I32)
        pltpu.sync_copy(gmax_hbm, gmax_v)

        def flush():
            for j in range(b // 16):
                psx[pl.ds(j * 16, 16)] = psrc[pl.ds(j * 16, 16)]
                plx[pl.ds(j * 16, 16)] = ploc[pl.ds(j * 16, 16)]
            pltpu.async_copy(g_hbm.at[psx], rows, sem).wait()

            def grp(eg, _):
                base = eg * 16
                locv = plsc.load_gather(plx, [iota + base])
                asr = plsc.load_gather(
                    rows, [iota + base, jnp.full((16,), acol, I32)])
                ad = plsc.load_gather(adst_v, [locv])
                ew = 1.0 / (1.0 + jnp.exp(-(asr + ad)))
                ewb[...] = ew
                cmb[...] = 1.0 - ew

                def edge(e2, _):
                    e = base + e2
                    ef = jnp.full((16,), e, I32)
                    locj = plsc.load_gather(plx, [ef])
                    ewv = plsc.load_gather(ewb, [jnp.full((16,), e2, I32)])
                    cmv = plsc.load_gather(cmb, [jnp.full((16,), e2, I32)])
                    for f in range(nf):
                        cvec = iota + f * 16
                        xj = plsc.load_gather(rows, [ef, cvec])
                        uu = plsc.load_gather(rows, [ef, cvec + s])
                        wn = plsc.load_gather(rows, [ef, cvec + 2 * s])
                        e1 = jnp.exp(wn - ewv * uu)
                        tt = 1.0 - cmv / (1.0 + e1)
                        msg = xj * tt
                        gm = gmax_v[pl.ds(f * 16, 16)]
                        ex = jnp.exp(msg - gm)
                        plsc.addupdate_scatter(acc, [locj, cvec], ex)
                        plsc.addupdate_scatter(acc, [locj, cvec + s], ex * msg)
                    return 0

                lax.fori_loop(0, 16, edge, 0)
                return 0

            lax.fori_loop(0, b // 16, grp, 0)

        def chunk(ch, _):
            lo = ch * crow + sid2 * own_n

            def zr(rr, _):
                rv = jnp.full((16,), rr, I32)
                for f in range(s2 // 16):
                    plsc.store_scatter(acc, [rv, iota + f * 16], zf)
                return 0

            lax.fori_loop(0, own_n + 1, zr, 0)
            pltpu.sync_copy(adst_hbm.at[pl.ds(lo, own_n)],
                            adst_v.at[pl.ds(0, own_n)])
            smem[0] = 0
            pltpu.sync_copy(src_hbm.at[pl.ds(0, eb)], esrc.at[pl.ds(0, eb)])
            pltpu.sync_copy(dst_hbm.at[pl.ds(0, eb)], edst.at[pl.ds(0, eb)])

            def stage(st, _):
                cb = lax.rem(st, 2) * eb
                nbb = (1 - lax.rem(st, 2)) * eb
                nxt = jnp.minimum(st + 1, nst - 1)
                hs = pltpu.async_copy(
                    src_hbm.at[pl.ds(nxt * eb, eb)],
                    esrc.at[pl.ds(nbb, eb)], sems)
                hd = pltpu.async_copy(
                    dst_hbm.at[pl.ds(nxt * eb, eb)],
                    edst.at[pl.ds(nbb, eb)], semd)

                def step(k, _):
                    vd = edst[pl.ds(cb + k * 16, 16)]
                    m = (vd >= lo) & (vd < lo + own_n)

                    @pl.when(jnp.any(m))
                    def _():
                        mi = m.astype(I32)
                        cnt = jnp.sum(mi)
                        vs = esrc[pl.ds(cb + k * 16, 16)]
                        np_ = smem[0]
                        csum = plsc.cumsum(mi)
                        pos = jnp.where(m, np_ + csum - 1, b + 16 + iota)
                        plsc.store_scatter(psrc, [pos], vs)
                        plsc.store_scatter(ploc, [pos], vd - lo)
                        smem[0] = np_ + cnt

                        @pl.when(np_ + cnt >= b)
                        def _():
                            flush()
                            psrc[pl.ds(0, 16)] = psrc[pl.ds(b, 16)]
                            ploc[pl.ds(0, 16)] = ploc[pl.ds(b, 16)]
                            smem[0] = np_ + cnt - b

                    return 0

                lax.fori_loop(0, eb // 16, step, 0)
                hs.wait()
                hd.wait()
                return 0

            lax.fori_loop(0, nst, stage, 0)

            npf = smem[0]

            @pl.when(npf > 0)
            def _():
                for j in range(0, b, 16):
                    cur = ploc[pl.ds(j, 16)]
                    pos = iota + j
                    ploc[pl.ds(j, 16)] = jnp.where(
                        pos >= npf, jnp.full((16,), dummy, I32), cur)
                flush()

            pltpu.sync_copy(acc.at[pl.ds(0, own_n)],
                            out_hbm.at[pl.ds(lo, own_n)])
            return 0

        lax.fori_loop(0, nch, chunk, 0)

    return krn


def _edge_plan(nd, s, e):
    s2 = 2 * s
    r = _ru(3 * s + 1, 128)
    b = 64
    eb = min(2048, max(16, _ru(e, 16)))
    nst = -(-e // eb)
    used = (b * r * 4 + 2 * 2 * eb * 4 + (b + 32) * 2 * 4 + 2 * b * 4
            + s * 4 + 256 + 16 * 1024)
    rem = 480 * 1024 - used
    max_own = ((rem - 16 * 4 - s2 * 4) // (s2 * 4 + 4)) // 16 * 16
    nch = max(1, -(-nd // (32 * max_own)))
    own_n = _ru(-(-nd // (32 * nch)), 16)
    return nch, own_n, eb, nst


def _fin_krn(s, has_ln, want_raw, acc_ref, xd_ref, w1a_ref, w1b_ref, b1_ref,
             w2_ref, b2_ref, g_ref, b_ref, *outs):
    acc = acc_ref[...]
    den = acc[:, :s]
    num = acc[:, s:]
    aggr = num / (den + 1e-16)
    h = (jnp.dot(aggr, w1a_ref[...], preferred_element_type=F32)
         + jnp.dot(xd_ref[...], w1b_ref[...], preferred_element_type=F32)
         + b1_ref[...])
    h = _mish(h)
    o = _mish(jnp.dot(h, w2_ref[...], preferred_element_type=F32) + b2_ref[...])
    k = 0
    if want_raw:
        outs[k][...] = o
        k += 1
    if has_ln:
        outs[k][...] = _ln_math(o, g_ref[...], b_ref[...])


def _finish(acc, xd, w1, b1, w2, b2, lnp=None, want_raw=True):
    nd, t = xd.shape
    s = acc.shape[1] // 2
    o = w2.shape[0]
    bn = min(512, _ru(nd, 8))
    grid = (pl.cdiv(nd, bn),)
    has_ln = lnp is not None
    g = lnp["g"] if has_ln else jnp.zeros((o,), F32)
    b = lnp["b"] if has_ln else jnp.zeros((o,), F32)
    out_shape = []
    out_specs = []
    if want_raw:
        out_shape.append(jax.ShapeDtypeStruct((nd, o), F32))
        out_specs.append(pl.BlockSpec((bn, o), lambda i: (i, 0)))
    if has_ln:
        out_shape.append(jax.ShapeDtypeStruct((nd, o), F32))
        out_specs.append(pl.BlockSpec((bn, o), lambda i: (i, 0)))
    res = pl.pallas_call(
        functools.partial(_fin_krn, s, has_ln, want_raw),
        grid=grid,
        in_specs=[
            pl.BlockSpec((bn, 2 * s), lambda i: (i, 0)),
            pl.BlockSpec((bn, t), lambda i: (i, 0)),
            pl.BlockSpec((s, o), lambda i: (0, 0)),
            pl.BlockSpec((t, o), lambda i: (0, 0)),
            pl.BlockSpec((o,), lambda i: (0,)),
            pl.BlockSpec((o, o), lambda i: (0, 0)),
            pl.BlockSpec((o,), lambda i: (0,)),
            pl.BlockSpec((o,), lambda i: (0,)),
            pl.BlockSpec((o,), lambda i: (0,)),
        ],
        out_specs=out_specs,
        out_shape=out_shape,
    )(acc, xd, w1[:s], w1[s:], b1, w2, b2, g, b)
    return res


def _xattn_krn(m_ctx, mb, inv_sqrt, x_ref, ctx_ref, wq_ref, bq_ref, wk_ref,
               bk_ref, wv_ref, bv_ref, wa_ref, ba_ref, wo_ref, bo_ref, g_ref,
               b_ref, o_ref, accs, mst, lst):
    j = pl.program_id(1)
    ncb = pl.num_programs(1)

    @pl.when(j == 0)
    def _():
        accs[...] = jnp.zeros_like(accs)
        mst[...] = jnp.full_like(mst, -1e30)
        lst[...] = jnp.zeros_like(lst)

    x = x_ref[...]
    ctx = ctx_ref[...]
    q = jnp.dot(x, wq_ref[...], preferred_element_type=F32) + bq_ref[...]
    kk = jnp.dot(ctx, wk_ref[...], preferred_element_type=F32) + bk_ref[...]
    vv = jnp.dot(ctx, wv_ref[...], preferred_element_type=F32) + bv_ref[...]
    sc = jnp.dot(q, kk.T, preferred_element_type=F32) * inv_sqrt
    cid = lax.broadcasted_iota(I32, sc.shape, 1) + j * mb
    sc = jnp.where(cid < m_ctx, sc, -1e30)
    m_old = mst[...]
    m_new = jnp.maximum(m_old, jnp.max(sc, axis=-1, keepdims=True))
    alpha = jnp.exp(m_old - m_new)
    p = jnp.exp(sc - m_new)
    lst[...] = lst[...] * alpha + jnp.sum(p, axis=-1, keepdims=True)
    accs[...] = accs[...] * alpha + jnp.dot(
        p, vv, preferred_element_type=F32)
    mst[...] = m_new

    @pl.when(j == ncb - 1)
    def _():
        out = accs[...] / lst[...]
        out = jnp.dot(out, wa_ref[...], preferred_element_type=F32) + ba_ref[...]
        out = jnp.dot(out, wo_ref[...], preferred_element_type=F32) + bo_ref[...]
        o_ref[...] = _ln_math(out + x, g_ref[...], b_ref[...])


def _xattn(x, ctx, p):
    n, qd = x.shape
    m, kd = ctx.shape
    bn = min(512, _ru(n, 8))
    mb = min(2048, _ru(m, 8))
    grid = (pl.cdiv(n, bn), pl.cdiv(m, mb))
    hh = 256
    krn = functools.partial(_xattn_krn, m, mb, 1.0 / math.sqrt(64.0))
    return pl.pallas_call(
        krn,
        grid=grid,
        in_specs=[
            pl.BlockSpec((bn, qd), lambda i, j: (i, 0)),
            pl.BlockSpec((mb, kd), lambda i, j: (j, 0)),
            pl.BlockSpec((qd, hh), lambda i, j: (0, 0)),
            pl.BlockSpec((hh,), lambda i, j: (0,)),
            pl.BlockSpec((kd, hh), lambda i, j: (0, 0)),
            pl.BlockSpec((hh,), lambda i, j: (0,)),
            pl.BlockSpec((kd, hh), lambda i, j: (0, 0)),
            pl.BlockSpec((hh,), lambda i, j: (0,)),
            pl.BlockSpec((hh, hh), lambda i, j: (0, 0)),
            pl.BlockSpec((hh,), lambda i, j: (0,)),
            pl.BlockSpec((hh, qd), lambda i, j: (0, 0)),
            pl.BlockSpec((qd,), lambda i, j: (0,)),
            pl.BlockSpec((qd,), lambda i, j: (0,)),
            pl.BlockSpec((qd,), lambda i, j: (0,)),
        ],
        out_specs=pl.BlockSpec((bn, qd), lambda i, j: (i, 0)),
        out_shape=jax.ShapeDtypeStruct((n, qd), F32),
        scratch_shapes=[
            pltpu.VMEM((bn, hh), F32),
            pltpu.VMEM((bn, 1), F32),
            pltpu.VMEM((bn, 1), F32),
        ],
    )(x, ctx, p["Wq"], p["bq"], p["Wk"], p["bk"], p["Wv"], p["bv"],
      p["Wa"], p["ba"], p["Wo"], p["bo"], p["g"], p["b"])


def _block_opt(x_src, x_dst, e, p, lnp=None, want_raw=True):
    ns, s = x_src.shape
    nd, t = x_dst.shape
    ee = e.shape[1]
    nch, own_n, eb, nst = _edge_plan(nd, s, ee)
    e_pad = eb * nst
    crow = 32 * own_n

    g_tab, gmax = _prep_src(x_src, p["Wg"], p["bg"], p["We"][t:])
    a_dst = _prep_dst(x_dst, p["We"][:t], p["be"])
    a_dst = jnp.pad(a_dst, (0, nch * crow - nd))
    src = jnp.pad(e[0], (0, e_pad - ee))
    dst = jnp.pad(e[1], (0, e_pad - ee), constant_values=-1)

    edge_fn = _make_edge_pass(ns, nd, s, nch, own_n, eb, nst)
    acc = edge_fn(g_tab, a_dst, gmax.reshape(-1), src, dst)
    acc = acc[:nd]
    return _finish(acc, x_dst, p["W1"], p["b1"], p["W2"], p["b2"],
                   lnp=lnp, want_raw=want_raw)


def kernel(x_hit, x_nexus, x_interaction, x_ophit, x_pmt, x_flash, e_plane,
           e_hit_nexus, e_nexus_interaction, e_interaction_nexus, e_nexus_hit,
           e_ophit_pmt, e_pmt_flash, e_flash_interaction, e_interaction_flash,
           e_flash_pmt, e_pmt_ophit, params):
    P = params
    x_hit = _xattn(x_hit, x_flash, P["hit_flash_attention"])
    x_nexus = _xattn(x_nexus, x_pmt, P["nexus_pmt_attention"])
    x_interaction = _xattn(x_interaction, x_ophit,
                           P["interaction_ophit_attention"])

    (x_hit,) = _block_opt(x_hit, x_hit, e_plane, P["plane_net"])
    (x_nexus,) = _block_opt(x_hit, x_nexus, e_hit_nexus, P["plane_to_nexus"])
    (x_interaction,) = _block_opt(x_nexus, x_interaction, e_nexus_interaction,
                                  P["nexus_to_interaction"])
    x_nexus, x_nexus_ln = _block_opt(x_interaction, x_nexus,
                                     e_interaction_nexus,
                                     P["interaction_to_nexus"],
                                     lnp=P["nexus_norm"])
    (x_hit_ln,) = _block_opt(x_nexus, x_hit, e_nexus_hit, P["nexus_to_plane"],
                             lnp=P["hit_norm"], want_raw=False)
    (x_pmt,) = _block_opt(x_ophit, x_pmt, e_ophit_pmt, P["ophit_to_pmt"])
    (x_flash,) = _block_opt(x_pmt, x_flash, e_pmt_flash, P["pmt_to_flash"])
    x_interaction, x_interaction_ln = _block_opt(
        x_flash, x_interaction, e_flash_interaction, P["flash_to_interaction"],
        lnp=P["interaction_norm"])
    x_flash, x_flash_ln = _block_opt(x_interaction, x_flash,
                                     e_interaction_flash,
                                     P["interaction_to_flash"],
                                     lnp=P["flash_norm"])
    x_pmt, x_pmt_ln = _block_opt(x_flash, x_pmt, e_flash_pmt,
                                 P["flash_to_pmt"], lnp=P["pmt_norm"])
    (x_ophit_ln,) = _block_opt(x_pmt, x_ophit, e_pmt_ophit, P["pmt_to_ophit"],
                               lnp=P["ophit_norm"], want_raw=False)

    return (x_hit_ln, x_nexus_ln, x_interaction_ln, x_ophit_ln, x_pmt_ln,
            x_flash_ln)


# speedup vs baseline: 1.3011x; 1.3011x over previous
import functools
import math

import jax
import jax.numpy as jnp
from jax import lax
from jax.experimental import pallas as pl
from jax.experimental.pallas import tpu as pltpu
from jax.experimental.pallas import tpu_sc as plsc

F32 = jnp.float32
I32 = jnp.int32


def _ru(x, m):
    return (x + m - 1) // m * m


def _ln_math(x, g, b):
    m = jnp.mean(x, -1, keepdims=True)
    v = jnp.mean((x - m) ** 2, -1, keepdims=True)
    return (x - m) / jnp.sqrt(v + 1e-5) * g + b


def _mish(x):
    return x * jnp.tanh(jax.nn.softplus(x))


def _prep_src_krn(ns, bn, s, r, x_ref, wg1_ref, wg2_ref, bg_ref, wes_ref,
                  g_ref, gmax_ref):
    i = pl.program_id(0)
    x = x_ref[...]
    u = jnp.dot(x, wg1_ref[...], preferred_element_type=F32)
    wneg = -(jnp.dot(x, wg2_ref[...], preferred_element_type=F32) + bg_ref[...])
    a = jnp.dot(x, wes_ref[...], preferred_element_type=F32)
    g_ref[:, 0:s] = x
    g_ref[:, s:2 * s] = u
    g_ref[:, 2 * s:3 * s] = wneg
    g_ref[:, 3 * s:r] = jnp.concatenate(
        [a, jnp.zeros((bn, r - 3 * s - 1), F32)], axis=1)
    rid = lax.broadcasted_iota(I32, (bn, 1), 0) + i * bn
    xm = jnp.where(rid < ns, x, -1e30)
    bmax = jnp.max(xm, axis=0, keepdims=True)

    @pl.when(i == 0)
    def _():
        gmax_ref[...] = jnp.zeros_like(gmax_ref)

    gmax_ref[...] = jnp.maximum(gmax_ref[...], bmax)


def _prep_src(x, wg, bg, wes):
    ns, s = x.shape
    r = _ru(3 * s + 1, 128)
    bn = 512
    grid = (pl.cdiv(ns, bn),)
    krn = functools.partial(_prep_src_krn, ns, bn, s, r)
    return pl.pallas_call(
        krn,
        grid=grid,
        in_specs=[
            pl.BlockSpec((bn, s), lambda i: (i, 0)),
            pl.BlockSpec((s, s), lambda i: (0, 0)),
            pl.BlockSpec((s, s), lambda i: (0, 0)),
            pl.BlockSpec((s,), lambda i: (0,)),
            pl.BlockSpec((s, 1), lambda i: (0, 0)),
        ],
        out_specs=[
            pl.BlockSpec((bn, r), lambda i: (i, 0)),
            pl.BlockSpec((1, s), lambda i: (0, 0)),
        ],
        out_shape=[
            jax.ShapeDtypeStruct((ns, r), F32),
            jax.ShapeDtypeStruct((1, s), F32),
        ],
    )(x, wg[:s], wg[s:], bg, wes)


def _adst_krn(x_ref, w_ref, be_ref, o_ref):
    o_ref[...] = jnp.dot(x_ref[...], w_ref[...],
                         preferred_element_type=F32) + be_ref[...]


def _prep_dst(x, wed, be):
    nd, t = x.shape
    bn = min(512, _ru(nd, 8))
    grid = (pl.cdiv(nd, bn),)
    out = pl.pallas_call(
        _adst_krn,
        grid=grid,
        in_specs=[
            pl.BlockSpec((bn, t), lambda i: (i, 0)),
            pl.BlockSpec((t, 1), lambda i: (0, 0)),
            pl.BlockSpec((1,), lambda i: (0,)),
        ],
        out_specs=pl.BlockSpec((bn, 1), lambda i: (i, 0)),
        out_shape=jax.ShapeDtypeStruct((nd, 1), F32),
    )(x, wed, be)
    return out.reshape(-1)


def _make_edge_pass(ns, nd, s, nch, own_n, eb, nst):
    s2 = 2 * s
    r = _ru(3 * s + 1, 128)
    acol = 3 * s
    nf = s // 16
    b = 64
    dummy = own_n
    crow = 32 * own_n
    mesh = plsc.VectorSubcoreMesh(core_axis_name="c", subcore_axis_name="s")

    @functools.partial(
        pl.kernel,
        mesh=mesh,
        compiler_params=pltpu.CompilerParams(needs_layout_passes=False),
        out_type=jax.ShapeDtypeStruct((nch * crow, s2), F32),
        scratch_types=[
            pltpu.VMEM((2 * eb,), I32),
            pltpu.VMEM((2 * eb,), I32),
            pltpu.VMEM((b + 32,), I32),
            pltpu.VMEM((b + 32,), I32),
            pltpu.VMEM((b,), I32),
            pltpu.VMEM((b,), I32),
            pltpu.VMEM((b, r), F32),
            pltpu.VMEM((own_n + 16,), F32),
            pltpu.VMEM((16,), F32),
            pltpu.VMEM((16,), F32),
            pltpu.VMEM((s,), F32),
            pltpu.VMEM((own_n + 1, s2), F32),
            pltpu.SMEM((4,), I32),
            pltpu.SemaphoreType.DMA,
            pltpu.SemaphoreType.DMA,
            pltpu.SemaphoreType.DMA,
        ],
    )
    def krn(g_hbm, adst_hbm, gmax_hbm, src_hbm, dst_hbm, out_hbm,
            esrc, edst, psrc, ploc, psx, plx, rows, adst_v, ewb, cmb,
            gmax_v, acc, smem, sem, sems, semd):
        cid = lax.axis_index("c")
        sid = lax.axis_index("s")
        sid2 = cid * 16 + sid
        iota = lax.iota(I32, 16)
        zi = jnp.zeros((16,), I32)
        zf = jnp.zeros((16,), F32)

        for j in range(0, b + 32, 16):
            psrc[pl.ds(j, 16)] = zi
            ploc[pl.ds(j, 16)] = jnp.full((16,), dummy, I32)
        pltpu.sync_copy(gmax_hbm, gmax_v)

        def flush():
            for j in range(b // 16):
                psx[pl.ds(j * 16, 16)] = psrc[pl.ds(j * 16, 16)]
                plx[pl.ds(j * 16, 16)] = ploc[pl.ds(j * 16, 16)]
            pltpu.async_copy(g_hbm.at[psx], rows, sem).wait()

            def grp(eg, _):
                base = eg * 16
                locv = plsc.load_gather(plx, [iota + base])
                asr = plsc.load_gather(
                    rows, [iota + base, jnp.full((16,), acol, I32)])
                ad = plsc.load_gather(adst_v, [locv])
                ew = 1.0 / (1.0 + jnp.exp(-(asr + ad)))
                ewb[...] = ew
                cmb[...] = 1.0 - ew

                def edge(e2, _):
                    e = base + e2
                    ef = jnp.full((16,), e, I32)
                    locj = plsc.load_gather(plx, [ef])
                    ewv = plsc.load_gather(ewb, [jnp.full((16,), e2, I32)])
                    cmv = plsc.load_gather(cmb, [jnp.full((16,), e2, I32)])
                    for f in range(nf):
                        cvec = iota + f * 16
                        xj = plsc.load_gather(rows, [ef, cvec])
                        uu = plsc.load_gather(rows, [ef, cvec + s])
                        wn = plsc.load_gather(rows, [ef, cvec + 2 * s])
                        e1 = jnp.exp(wn - ewv * uu)
                        tt = 1.0 - cmv / (1.0 + e1)
                        msg = xj * tt
                        gm = gmax_v[pl.ds(f * 16, 16)]
                        ex = jnp.exp(msg - gm)
                        plsc.addupdate_scatter(acc, [locj, cvec], ex)
                        plsc.addupdate_scatter(acc, [locj, cvec + s], ex * msg)
                    return 0

                lax.fori_loop(0, 16, edge, 0)
                return 0

            lax.fori_loop(0, b // 16, grp, 0)

        def chunk(ch, _):
            lo = ch * crow + sid2 * own_n

            def zr(rr, _):
                rv = jnp.full((16,), rr, I32)
                for f in range(s2 // 16):
                    plsc.store_scatter(acc, [rv, iota + f * 16], zf)
                return 0

            lax.fori_loop(0, own_n + 1, zr, 0)
            pltpu.sync_copy(adst_hbm.at[pl.ds(lo, own_n)],
                            adst_v.at[pl.ds(0, own_n)])
            smem[0] = 0
            pltpu.sync_copy(src_hbm.at[pl.ds(0, eb)], esrc.at[pl.ds(0, eb)])
            pltpu.sync_copy(dst_hbm.at[pl.ds(0, eb)], edst.at[pl.ds(0, eb)])

            def stage(st, _):
                cb = lax.rem(st, 2) * eb
                nbb = (1 - lax.rem(st, 2)) * eb
                nxt = jnp.minimum(st + 1, nst - 1)
                hs = pltpu.async_copy(
                    src_hbm.at[pl.ds(nxt * eb, eb)],
                    esrc.at[pl.ds(nbb, eb)], sems)
                hd = pltpu.async_copy(
                    dst_hbm.at[pl.ds(nxt * eb, eb)],
                    edst.at[pl.ds(nbb, eb)], semd)

                def step(k, _):
                    vd = edst[pl.ds(cb + k * 16, 16)]
                    m = (vd >= lo) & (vd < lo + own_n)
                    mi = m.astype(I32)
                    cnt = jnp.sum(mi)

                    @pl.when(cnt > 0)
                    def _():
                        vs = esrc[pl.ds(cb + k * 16, 16)]
                        np_ = smem[0]
                        csum = plsc.cumsum(mi)
                        pos = jnp.where(m, np_ + csum - 1, b + 16 + iota)
                        plsc.store_scatter(psrc, [pos], vs)
                        plsc.store_scatter(ploc, [pos], vd - lo)
                        smem[0] = np_ + cnt

                        @pl.when(np_ + cnt >= b)
                        def _():
                            flush()
                            psrc[pl.ds(0, 16)] = psrc[pl.ds(b, 16)]
                            ploc[pl.ds(0, 16)] = ploc[pl.ds(b, 16)]
                            smem[0] = np_ + cnt - b

                    return 0

                lax.fori_loop(0, eb // 16, step, 0)
                hs.wait()
                hd.wait()
                return 0

            lax.fori_loop(0, nst, stage, 0)

            npf = smem[0]

            @pl.when(npf > 0)
            def _():
                for j in range(0, b, 16):
                    cur = ploc[pl.ds(j, 16)]
                    pos = iota + j
                    ploc[pl.ds(j, 16)] = jnp.where(
                        pos >= npf, jnp.full((16,), dummy, I32), cur)
                flush()

            pltpu.sync_copy(acc.at[pl.ds(0, own_n)],
                            out_hbm.at[pl.ds(lo, own_n)])
            return 0

        lax.fori_loop(0, nch, chunk, 0)

    return krn


def _edge_plan(nd, s, e):
    s2 = 2 * s
    r = _ru(3 * s + 1, 128)
    b = 64
    eb = min(1024, max(16, _ru(e, 16)))
    nst = -(-e // eb)
    used = (b * r * 4 + 2 * 2 * eb * 4 + (b + 32) * 2 * 4 + 2 * b * 4
            + s * 4 + 256 + 8 * 1024)
    rem = 480 * 1024 - used
    max_own = ((rem - 16 * 4 - s2 * 4) // (s2 * 4 + 4)) // 16 * 16
    nch = max(1, -(-nd // (32 * max_own)))
    own_n = _ru(-(-nd // (32 * nch)), 16)
    return nch, own_n, eb, nst


def _fin_krn(s, has_ln, want_raw, acc_ref, xd_ref, w1a_ref, w1b_ref, b1_ref,
             w2_ref, b2_ref, g_ref, b_ref, *outs):
    acc = acc_ref[...]
    den = acc[:, :s]
    num = acc[:, s:]
    aggr = num / (den + 1e-16)
    h = (jnp.dot(aggr, w1a_ref[...], preferred_element_type=F32)
         + jnp.dot(xd_ref[...], w1b_ref[...], preferred_element_type=F32)
         + b1_ref[...])
    h = _mish(h)
    o = _mish(jnp.dot(h, w2_ref[...], preferred_element_type=F32) + b2_ref[...])
    k = 0
    if want_raw:
        outs[k][...] = o
        k += 1
    if has_ln:
        outs[k][...] = _ln_math(o, g_ref[...], b_ref[...])


def _finish(acc, xd, w1, b1, w2, b2, lnp=None, want_raw=True):
    nd, t = xd.shape
    s = acc.shape[1] // 2
    o = w2.shape[0]
    bn = min(512, _ru(nd, 8))
    grid = (pl.cdiv(nd, bn),)
    has_ln = lnp is not None
    g = lnp["g"] if has_ln else jnp.zeros((o,), F32)
    b = lnp["b"] if has_ln else jnp.zeros((o,), F32)
    out_shape = []
    out_specs = []
    if want_raw:
        out_shape.append(jax.ShapeDtypeStruct((nd, o), F32))
        out_specs.append(pl.BlockSpec((bn, o), lambda i: (i, 0)))
    if has_ln:
        out_shape.append(jax.ShapeDtypeStruct((nd, o), F32))
        out_specs.append(pl.BlockSpec((bn, o), lambda i: (i, 0)))
    res = pl.pallas_call(
        functools.partial(_fin_krn, s, has_ln, want_raw),
        grid=grid,
        in_specs=[
            pl.BlockSpec((bn, 2 * s), lambda i: (i, 0)),
            pl.BlockSpec((bn, t), lambda i: (i, 0)),
            pl.BlockSpec((s, o), lambda i: (0, 0)),
            pl.BlockSpec((t, o), lambda i: (0, 0)),
            pl.BlockSpec((o,), lambda i: (0,)),
            pl.BlockSpec((o, o), lambda i: (0, 0)),
            pl.BlockSpec((o,), lambda i: (0,)),
            pl.BlockSpec((o,), lambda i: (0,)),
            pl.BlockSpec((o,), lambda i: (0,)),
        ],
        out_specs=out_specs,
        out_shape=out_shape,
    )(acc, xd, w1[:s], w1[s:], b1, w2, b2, g, b)
    return res


def _xattn_krn(m_ctx, mb, inv_sqrt, x_ref, ctx_ref, wq_ref, bq_ref, wk_ref,
               bk_ref, wv_ref, bv_ref, wa_ref, ba_ref, wo_ref, bo_ref, g_ref,
               b_ref, o_ref, accs, mst, lst):
    j = pl.program_id(1)
    ncb = pl.num_programs(1)

    @pl.when(j == 0)
    def _():
        accs[...] = jnp.zeros_like(accs)
        mst[...] = jnp.full_like(mst, -1e30)
        lst[...] = jnp.zeros_like(lst)

    x = x_ref[...]
    ctx = ctx_ref[...]
    q = jnp.dot(x, wq_ref[...], preferred_element_type=F32) + bq_ref[...]
    kk = jnp.dot(ctx, wk_ref[...], preferred_element_type=F32) + bk_ref[...]
    vv = jnp.dot(ctx, wv_ref[...], preferred_element_type=F32) + bv_ref[...]
    sc = jnp.dot(q, kk.T, preferred_element_type=F32) * inv_sqrt
    cid = lax.broadcasted_iota(I32, sc.shape, 1) + j * mb
    sc = jnp.where(cid < m_ctx, sc, -1e30)
    m_old = mst[...]
    m_new = jnp.maximum(m_old, jnp.max(sc, axis=-1, keepdims=True))
    alpha = jnp.exp(m_old - m_new)
    p = jnp.exp(sc - m_new)
    lst[...] = lst[...] * alpha + jnp.sum(p, axis=-1, keepdims=True)
    accs[...] = accs[...] * alpha + jnp.dot(
        p, vv, preferred_element_type=F32)
    mst[...] = m_new

    @pl.when(j == ncb - 1)
    def _():
        out = accs[...] / lst[...]
        out = jnp.dot(out, wa_ref[...], preferred_element_type=F32) + ba_ref[...]
        out = jnp.dot(out, wo_ref[...], preferred_element_type=F32) + bo_ref[...]
        o_ref[...] = _ln_math(out + x, g_ref[...], b_ref[...])


def _xattn(x, ctx, p):
    n, qd = x.shape
    m, kd = ctx.shape
    bn = min(512, _ru(n, 8))
    mb = min(2048, _ru(m, 8))
    grid = (pl.cdiv(n, bn), pl.cdiv(m, mb))
    hh = 256
    krn = functools.partial(_xattn_krn, m, mb, 1.0 / math.sqrt(64.0))
    return pl.pallas_call(
        krn,
        grid=grid,
        in_specs=[
            pl.BlockSpec((bn, qd), lambda i, j: (i, 0)),
            pl.BlockSpec((mb, kd), lambda i, j: (j, 0)),
            pl.BlockSpec((qd, hh), lambda i, j: (0, 0)),
            pl.BlockSpec((hh,), lambda i, j: (0,)),
            pl.BlockSpec((kd, hh), lambda i, j: (0, 0)),
            pl.BlockSpec((hh,), lambda i, j: (0,)),
            pl.BlockSpec((kd, hh), lambda i, j: (0, 0)),
            pl.BlockSpec((hh,), lambda i, j: (0,)),
            pl.BlockSpec((hh, hh), lambda i, j: (0, 0)),
            pl.BlockSpec((hh,), lambda i, j: (0,)),
            pl.BlockSpec((hh, qd), lambda i, j: (0, 0)),
            pl.BlockSpec((qd,), lambda i, j: (0,)),
            pl.BlockSpec((qd,), lambda i, j: (0,)),
            pl.BlockSpec((qd,), lambda i, j: (0,)),
        ],
        out_specs=pl.BlockSpec((bn, qd), lambda i, j: (i, 0)),
        out_shape=jax.ShapeDtypeStruct((n, qd), F32),
        scratch_shapes=[
            pltpu.VMEM((bn, hh), F32),
            pltpu.VMEM((bn, 1), F32),
            pltpu.VMEM((bn, 1), F32),
        ],
    )(x, ctx, p["Wq"], p["bq"], p["Wk"], p["bk"], p["Wv"], p["bv"],
      p["Wa"], p["ba"], p["Wo"], p["bo"], p["g"], p["b"])


def _block_opt(x_src, x_dst, e, p, lnp=None, want_raw=True):
    ns, s = x_src.shape
    nd, t = x_dst.shape
    ee = e.shape[1]
    nch, own_n, eb, nst = _edge_plan(nd, s, ee)
    e_pad = eb * nst
    crow = 32 * own_n

    g_tab, gmax = _prep_src(x_src, p["Wg"], p["bg"], p["We"][t:])
    a_dst = _prep_dst(x_dst, p["We"][:t], p["be"])
    a_dst = jnp.pad(a_dst, (0, nch * crow - nd))
    src = jnp.pad(e[0], (0, e_pad - ee))
    dst = jnp.pad(e[1], (0, e_pad - ee), constant_values=-1)

    edge_fn = _make_edge_pass(ns, nd, s, nch, own_n, eb, nst)
    acc = edge_fn(g_tab, a_dst, gmax.reshape(-1), src, dst)
    acc = acc[:nd]
    return _finish(acc, x_dst, p["W1"], p["b1"], p["W2"], p["b2"],
                   lnp=lnp, want_raw=want_raw)


def kernel(x_hit, x_nexus, x_interaction, x_ophit, x_pmt, x_flash, e_plane,
           e_hit_nexus, e_nexus_interaction, e_interaction_nexus, e_nexus_hit,
           e_ophit_pmt, e_pmt_flash, e_flash_interaction, e_interaction_flash,
           e_flash_pmt, e_pmt_ophit, params):
    P = params
    x_hit = _xattn(x_hit, x_flash, P["hit_flash_attention"])
    x_nexus = _xattn(x_nexus, x_pmt, P["nexus_pmt_attention"])
    x_interaction = _xattn(x_interaction, x_ophit,
                           P["interaction_ophit_attention"])

    (x_hit,) = _block_opt(x_hit, x_hit, e_plane, P["plane_net"])
    (x_nexus,) = _block_opt(x_hit, x_nexus, e_hit_nexus, P["plane_to_nexus"])
    (x_interaction,) = _block_opt(x_nexus, x_interaction, e_nexus_interaction,
                                  P["nexus_to_interaction"])
    x_nexus, x_nexus_ln = _block_opt(x_interaction, x_nexus,
                                     e_interaction_nexus,
                                     P["interaction_to_nexus"],
                                     lnp=P["nexus_norm"])
    (x_hit_ln,) = _block_opt(x_nexus, x_hit, e_nexus_hit, P["nexus_to_plane"],
                             lnp=P["hit_norm"], want_raw=False)
    (x_pmt,) = _block_opt(x_ophit, x_pmt, e_ophit_pmt, P["ophit_to_pmt"])
    (x_flash,) = _block_opt(x_pmt, x_flash, e_pmt_flash, P["pmt_to_flash"])
    x_interaction, x_interaction_ln = _block_opt(
        x_flash, x_interaction, e_flash_interaction, P["flash_to_interaction"],
        lnp=P["interaction_norm"])
    x_flash, x_flash_ln = _block_opt(x_interaction, x_flash,
                                     e_interaction_flash,
                                     P["interaction_to_flash"],
                                     lnp=P["flash_norm"])
    x_pmt, x_pmt_ln = _block_opt(x_flash, x_pmt, e_flash_pmt,
                                 P["flash_to_pmt"], lnp=P["pmt_norm"])
    (x_ophit_ln,) = _block_opt(x_pmt, x_ophit, e_pmt_ophit, P["pmt_to_ophit"],
                               lnp=P["ophit_norm"], want_raw=False)

    return (x_hit_ln, x_nexus_ln, x_interaction_ln, x_ophit_ln, x_pmt_ln,
            x_flash_ln)


# speedup vs baseline: 1.6201x; 1.2452x over previous
import functools
import math

import jax
import jax.numpy as jnp
from jax import lax
from jax.experimental import pallas as pl
from jax.experimental.pallas import tpu as pltpu
from jax.experimental.pallas import tpu_sc as plsc

F32 = jnp.float32
I32 = jnp.int32


def _ru(x, m):
    return (x + m - 1) // m * m


def _ln_math(x, g, b):
    m = jnp.mean(x, -1, keepdims=True)
    v = jnp.mean((x - m) ** 2, -1, keepdims=True)
    return (x - m) / jnp.sqrt(v + 1e-5) * g + b


def _mish(x):
    return x * jnp.tanh(jax.nn.softplus(x))


def _prep_src_krn(ns, bn, s, r, x_ref, wg1_ref, wg2_ref, bg_ref, wes_ref,
                  g_ref, gmax_ref):
    i = pl.program_id(0)
    x = x_ref[...]
    u = jnp.dot(x, wg1_ref[...], preferred_element_type=F32)
    wneg = -(jnp.dot(x, wg2_ref[...], preferred_element_type=F32) + bg_ref[...])
    a = jnp.dot(x, wes_ref[...], preferred_element_type=F32)
    g_ref[:, 0:s] = x
    g_ref[:, s:2 * s] = u
    g_ref[:, 2 * s:3 * s] = wneg
    g_ref[:, 3 * s:r] = jnp.concatenate(
        [a, jnp.zeros((bn, r - 3 * s - 1), F32)], axis=1)
    rid = lax.broadcasted_iota(I32, (bn, 1), 0) + i * bn
    xm = jnp.where(rid < ns, x, -1e30)
    bmax = jnp.max(xm, axis=0, keepdims=True)

    @pl.when(i == 0)
    def _():
        gmax_ref[...] = jnp.zeros_like(gmax_ref)

    gmax_ref[...] = jnp.maximum(gmax_ref[...], bmax)


def _prep_src(x, wg, bg, wes):
    ns, s = x.shape
    r = _ru(3 * s + 1, 128)
    bn = 512
    grid = (pl.cdiv(ns, bn),)
    krn = functools.partial(_prep_src_krn, ns, bn, s, r)
    return pl.pallas_call(
        krn,
        grid=grid,
        in_specs=[
            pl.BlockSpec((bn, s), lambda i: (i, 0)),
            pl.BlockSpec((s, s), lambda i: (0, 0)),
            pl.BlockSpec((s, s), lambda i: (0, 0)),
            pl.BlockSpec((s,), lambda i: (0,)),
            pl.BlockSpec((s, 1), lambda i: (0, 0)),
        ],
        out_specs=[
            pl.BlockSpec((bn, r), lambda i: (i, 0)),
            pl.BlockSpec((1, s), lambda i: (0, 0)),
        ],
        out_shape=[
            jax.ShapeDtypeStruct((ns, r), F32),
            jax.ShapeDtypeStruct((1, s), F32),
        ],
    )(x, wg[:s], wg[s:], bg, wes)


def _adst_krn(x_ref, w_ref, be_ref, o_ref):
    o_ref[...] = jnp.dot(x_ref[...], w_ref[...],
                         preferred_element_type=F32) + be_ref[...]


def _prep_dst(x, wed, be):
    nd, t = x.shape
    bn = min(512, _ru(nd, 8))
    grid = (pl.cdiv(nd, bn),)
    out = pl.pallas_call(
        _adst_krn,
        grid=grid,
        in_specs=[
            pl.BlockSpec((bn, t), lambda i: (i, 0)),
            pl.BlockSpec((t, 1), lambda i: (0, 0)),
            pl.BlockSpec((1,), lambda i: (0,)),
        ],
        out_specs=pl.BlockSpec((bn, 1), lambda i: (i, 0)),
        out_shape=jax.ShapeDtypeStruct((nd, 1), F32),
    )(x, wed, be)
    return out.reshape(-1)


def _make_edge_pass(ns, nd, s, nch, own_n, eb, nst):
    s2 = 2 * s
    r = _ru(3 * s + 1, 128)
    acol = 3 * s
    nf = s // 16
    b = 64
    dummy = own_n
    crow = 32 * own_n
    mesh = plsc.VectorSubcoreMesh(core_axis_name="c", subcore_axis_name="s")

    @functools.partial(
        pl.kernel,
        mesh=mesh,
        compiler_params=pltpu.CompilerParams(needs_layout_passes=False),
        out_type=jax.ShapeDtypeStruct((nch * crow, s2), F32),
        scratch_types=[
            pltpu.VMEM((2 * eb,), I32),
            pltpu.VMEM((2 * eb,), I32),
            pltpu.VMEM((b + 64,), I32),
            pltpu.VMEM((b + 64,), I32),
            pltpu.VMEM((b,), I32),
            pltpu.VMEM((b,), I32),
            pltpu.VMEM((b, r), F32),
            pltpu.VMEM((own_n + 16,), F32),
            pltpu.VMEM((16,), F32),
            pltpu.VMEM((16,), F32),
            pltpu.VMEM((s,), F32),
            pltpu.VMEM((own_n + 1, s2), F32),
            pltpu.SMEM((4,), I32),
            pltpu.SemaphoreType.DMA,
            pltpu.SemaphoreType.DMA,
            pltpu.SemaphoreType.DMA,
        ],
    )
    def krn(g_hbm, adst_hbm, gmax_hbm, src_hbm, dst_hbm, out_hbm,
            esrc, edst, psrc, ploc, psx, plx, rows, adst_v, ewb, cmb,
            gmax_v, acc, smem, sem, sems, semd):
        cid = lax.axis_index("c")
        sid = lax.axis_index("s")
        sid2 = cid * 16 + sid
        iota = lax.iota(I32, 16)
        zi = jnp.zeros((16,), I32)
        zf = jnp.zeros((16,), F32)

        for j in range(0, b + 64, 16):
            psrc[pl.ds(j, 16)] = zi
            ploc[pl.ds(j, 16)] = jnp.full((16,), dummy, I32)
        pltpu.sync_copy(gmax_hbm, gmax_v)

        def flush():
            for j in range(b // 16):
                psx[pl.ds(j * 16, 16)] = psrc[pl.ds(j * 16, 16)]
                plx[pl.ds(j * 16, 16)] = ploc[pl.ds(j * 16, 16)]
            pltpu.async_copy(g_hbm.at[psx], rows, sem).wait()

            def grp(eg, _):
                base = eg * 16
                locv = plsc.load_gather(plx, [iota + base])
                asr = plsc.load_gather(
                    rows, [iota + base, jnp.full((16,), acol, I32)])
                ad = plsc.load_gather(adst_v, [locv])
                ew = 1.0 / (1.0 + jnp.exp(-(asr + ad)))
                ewb[...] = ew
                cmb[...] = 1.0 - ew

                def edge(e2, _):
                    e = base + e2
                    ef = jnp.full((16,), e, I32)
                    locj = plsc.load_gather(plx, [ef])
                    ewv = plsc.load_gather(ewb, [jnp.full((16,), e2, I32)])
                    cmv = plsc.load_gather(cmb, [jnp.full((16,), e2, I32)])
                    for f in range(nf):
                        cvec = iota + f * 16
                        xj = plsc.load_gather(rows, [ef, cvec])
                        uu = plsc.load_gather(rows, [ef, cvec + s])
                        wn = plsc.load_gather(rows, [ef, cvec + 2 * s])
                        e1 = jnp.exp(wn - ewv * uu)
                        tt = 1.0 - cmv / (1.0 + e1)
                        msg = xj * tt
                        gm = gmax_v[pl.ds(f * 16, 16)]
                        ex = jnp.exp(msg - gm)
                        plsc.addupdate_scatter(acc, [locj, cvec], ex)
                        plsc.addupdate_scatter(acc, [locj, cvec + s], ex * msg)
                    return 0

                lax.fori_loop(0, 16, edge, 0)
                return 0

            lax.fori_loop(0, b // 16, grp, 0)

        def chunk(ch, _):
            lo = ch * crow + sid2 * own_n

            def zr(rr, _):
                rv = jnp.full((16,), rr, I32)
                for f in range(s2 // 16):
                    plsc.store_scatter(acc, [rv, iota + f * 16], zf)
                return 0

            lax.fori_loop(0, own_n + 1, zr, 0)
            pltpu.sync_copy(adst_hbm.at[pl.ds(lo, own_n)],
                            adst_v.at[pl.ds(0, own_n)])
            smem[0] = 0
            pltpu.sync_copy(src_hbm.at[pl.ds(0, eb)], esrc.at[pl.ds(0, eb)])
            pltpu.sync_copy(dst_hbm.at[pl.ds(0, eb)], edst.at[pl.ds(0, eb)])

            def stage(st, _):
                cb = lax.rem(st, 2) * eb
                nbb = (1 - lax.rem(st, 2)) * eb
                nxt = jnp.minimum(st + 1, nst - 1)
                hs = pltpu.async_copy(
                    src_hbm.at[pl.ds(nxt * eb, eb)],
                    esrc.at[pl.ds(nbb, eb)], sems)
                hd = pltpu.async_copy(
                    dst_hbm.at[pl.ds(nxt * eb, eb)],
                    edst.at[pl.ds(nbb, eb)], semd)

                def step(k, _):
                    vd0 = edst[pl.ds(cb + k * 32, 16)]
                    vd1 = edst[pl.ds(cb + k * 32 + 16, 16)]
                    m0 = (vd0 >= lo) & (vd0 < lo + own_n)
                    m1 = (vd1 >= lo) & (vd1 < lo + own_n)
                    mi0 = m0.astype(I32)
                    mi1 = m1.astype(I32)
                    cnt = jnp.sum(mi0 + mi1)

                    @pl.when(cnt > 0)
                    def _():
                        vs0 = esrc[pl.ds(cb + k * 32, 16)]
                        vs1 = esrc[pl.ds(cb + k * 32 + 16, 16)]
                        np_ = smem[0]
                        c0s = plsc.all_reduce_population_count(m0)
                        pos0 = jnp.where(
                            m0, np_ + plsc.cumsum(mi0) - 1, b + 32 + iota)
                        pos1 = jnp.where(
                            m1, np_ + c0s + plsc.cumsum(mi1) - 1,
                            b + 32 + iota)
                        plsc.store_scatter(psrc, [pos0], vs0)
                        plsc.store_scatter(ploc, [pos0], vd0 - lo)
                        plsc.store_scatter(psrc, [pos1], vs1)
                        plsc.store_scatter(ploc, [pos1], vd1 - lo)
                        smem[0] = np_ + cnt

                        @pl.when(np_ + cnt >= b)
                        def _():
                            flush()
                            psrc[pl.ds(0, 16)] = psrc[pl.ds(b, 16)]
                            ploc[pl.ds(0, 16)] = ploc[pl.ds(b, 16)]
                            psrc[pl.ds(16, 16)] = psrc[pl.ds(b + 16, 16)]
                            ploc[pl.ds(16, 16)] = ploc[pl.ds(b + 16, 16)]
                            smem[0] = np_ + cnt - b

                    return 0

                lax.fori_loop(0, eb // 32, step, 0)
                hs.wait()
                hd.wait()
                return 0

            lax.fori_loop(0, nst, stage, 0)

            npf = smem[0]

            @pl.when(npf > 0)
            def _():
                for j in range(0, b, 16):
                    cur = ploc[pl.ds(j, 16)]
                    pos = iota + j
                    ploc[pl.ds(j, 16)] = jnp.where(
                        pos >= npf, jnp.full((16,), dummy, I32), cur)
                flush()

            pltpu.sync_copy(acc.at[pl.ds(0, own_n)],
                            out_hbm.at[pl.ds(lo, own_n)])
            return 0

        lax.fori_loop(0, nch, chunk, 0)

    return krn


def _edge_plan(nd, s, e):
    s2 = 2 * s
    r = _ru(3 * s + 1, 128)
    b = 64
    eb = min(1024, max(32, _ru(e, 32)))
    nst = -(-e // eb)
    used = (b * r * 4 + 2 * 2 * eb * 4 + (b + 64) * 2 * 4 + 2 * b * 4
            + s * 4 + 256 + 8 * 1024)
    rem = 480 * 1024 - used
    max_own = ((rem - 16 * 4 - s2 * 4) // (s2 * 4 + 4)) // 16 * 16
    nch = max(1, -(-nd // (32 * max_own)))
    own_n = _ru(-(-nd // (32 * nch)), 16)
    return nch, own_n, eb, nst


def _fin_krn(s, has_ln, want_raw, acc_ref, xd_ref, w1a_ref, w1b_ref, b1_ref,
             w2_ref, b2_ref, g_ref, b_ref, *outs):
    acc = acc_ref[...]
    den = acc[:, :s]
    num = acc[:, s:]
    aggr = num / (den + 1e-16)
    h = (jnp.dot(aggr, w1a_ref[...], preferred_element_type=F32)
         + jnp.dot(xd_ref[...], w1b_ref[...], preferred_element_type=F32)
         + b1_ref[...])
    h = _mish(h)
    o = _mish(jnp.dot(h, w2_ref[...], preferred_element_type=F32) + b2_ref[...])
    k = 0
    if want_raw:
        outs[k][...] = o
        k += 1
    if has_ln:
        outs[k][...] = _ln_math(o, g_ref[...], b_ref[...])


def _finish(acc, xd, w1, b1, w2, b2, lnp=None, want_raw=True):
    nd, t = xd.shape
    s = acc.shape[1] // 2
    o = w2.shape[0]
    bn = min(512, _ru(nd, 8))
    grid = (pl.cdiv(nd, bn),)
    has_ln = lnp is not None
    g = lnp["g"] if has_ln else jnp.zeros((o,), F32)
    b = lnp["b"] if has_ln else jnp.zeros((o,), F32)
    out_shape = []
    out_specs = []
    if want_raw:
        out_shape.append(jax.ShapeDtypeStruct((nd, o), F32))
        out_specs.append(pl.BlockSpec((bn, o), lambda i: (i, 0)))
    if has_ln:
        out_shape.append(jax.ShapeDtypeStruct((nd, o), F32))
        out_specs.append(pl.BlockSpec((bn, o), lambda i: (i, 0)))
    res = pl.pallas_call(
        functools.partial(_fin_krn, s, has_ln, want_raw),
        grid=grid,
        in_specs=[
            pl.BlockSpec((bn, 2 * s), lambda i: (i, 0)),
            pl.BlockSpec((bn, t), lambda i: (i, 0)),
            pl.BlockSpec((s, o), lambda i: (0, 0)),
            pl.BlockSpec((t, o), lambda i: (0, 0)),
            pl.BlockSpec((o,), lambda i: (0,)),
            pl.BlockSpec((o, o), lambda i: (0, 0)),
            pl.BlockSpec((o,), lambda i: (0,)),
            pl.BlockSpec((o,), lambda i: (0,)),
            pl.BlockSpec((o,), lambda i: (0,)),
        ],
        out_specs=out_specs,
        out_shape=out_shape,
    )(acc, xd, w1[:s], w1[s:], b1, w2, b2, g, b)
    return res


def _xattn_krn(m_ctx, mb, inv_sqrt, x_ref, ctx_ref, wq_ref, bq_ref, wk_ref,
               bk_ref, wv_ref, bv_ref, wa_ref, ba_ref, wo_ref, bo_ref, g_ref,
               b_ref, o_ref, accs, mst, lst):
    j = pl.program_id(1)
    ncb = pl.num_programs(1)

    @pl.when(j == 0)
    def _():
        accs[...] = jnp.zeros_like(accs)
        mst[...] = jnp.full_like(mst, -1e30)
        lst[...] = jnp.zeros_like(lst)

    x = x_ref[...]
    ctx = ctx_ref[...]
    q = jnp.dot(x, wq_ref[...], preferred_element_type=F32) + bq_ref[...]
    kk = jnp.dot(ctx, wk_ref[...], preferred_element_type=F32) + bk_ref[...]
    vv = jnp.dot(ctx, wv_ref[...], preferred_element_type=F32) + bv_ref[...]
    sc = jnp.dot(q, kk.T, preferred_element_type=F32) * inv_sqrt
    cid = lax.broadcasted_iota(I32, sc.shape, 1) + j * mb
    sc = jnp.where(cid < m_ctx, sc, -1e30)
    m_old = mst[...]
    m_new = jnp.maximum(m_old, jnp.max(sc, axis=-1, keepdims=True))
    alpha = jnp.exp(m_old - m_new)
    p = jnp.exp(sc - m_new)
    lst[...] = lst[...] * alpha + jnp.sum(p, axis=-1, keepdims=True)
    accs[...] = accs[...] * alpha + jnp.dot(
        p, vv, preferred_element_type=F32)
    mst[...] = m_new

    @pl.when(j == ncb - 1)
    def _():
        out = accs[...] / lst[...]
        out = jnp.dot(out, wa_ref[...], preferred_element_type=F32) + ba_ref[...]
        out = jnp.dot(out, wo_ref[...], preferred_element_type=F32) + bo_ref[...]
        o_ref[...] = _ln_math(out + x, g_ref[...], b_ref[...])


def _xattn(x, ctx, p):
    n, qd = x.shape
    m, kd = ctx.shape
    bn = min(512, _ru(n, 8))
    mb = min(2048, _ru(m, 8))
    grid = (pl.cdiv(n, bn), pl.cdiv(m, mb))
    hh = 256
    krn = functools.partial(_xattn_krn, m, mb, 1.0 / math.sqrt(64.0))
    return pl.pallas_call(
        krn,
        grid=grid,
        in_specs=[
            pl.BlockSpec((bn, qd), lambda i, j: (i, 0)),
            pl.BlockSpec((mb, kd), lambda i, j: (j, 0)),
            pl.BlockSpec((qd, hh), lambda i, j: (0, 0)),
            pl.BlockSpec((hh,), lambda i, j: (0,)),
            pl.BlockSpec((kd, hh), lambda i, j: (0, 0)),
            pl.BlockSpec((hh,), lambda i, j: (0,)),
            pl.BlockSpec((kd, hh), lambda i, j: (0, 0)),
            pl.BlockSpec((hh,), lambda i, j: (0,)),
            pl.BlockSpec((hh, hh), lambda i, j: (0, 0)),
            pl.BlockSpec((hh,), lambda i, j: (0,)),
            pl.BlockSpec((hh, qd), lambda i, j: (0, 0)),
            pl.BlockSpec((qd,), lambda i, j: (0,)),
            pl.BlockSpec((qd,), lambda i, j: (0,)),
            pl.BlockSpec((qd,), lambda i, j: (0,)),
        ],
        out_specs=pl.BlockSpec((bn, qd), lambda i, j: (i, 0)),
        out_shape=jax.ShapeDtypeStruct((n, qd), F32),
        scratch_shapes=[
            pltpu.VMEM((bn, hh), F32),
            pltpu.VMEM((bn, 1), F32),
            pltpu.VMEM((bn, 1), F32),
        ],
    )(x, ctx, p["Wq"], p["bq"], p["Wk"], p["bk"], p["Wv"], p["bv"],
      p["Wa"], p["ba"], p["Wo"], p["bo"], p["g"], p["b"])


def _block_opt(x_src, x_dst, e, p, lnp=None, want_raw=True):
    ns, s = x_src.shape
    nd, t = x_dst.shape
    ee = e.shape[1]
    nch, own_n, eb, nst = _edge_plan(nd, s, ee)
    e_pad = eb * nst
    crow = 32 * own_n

    g_tab, gmax = _prep_src(x_src, p["Wg"], p["bg"], p["We"][t:])
    a_dst = _prep_dst(x_dst, p["We"][:t], p["be"])
    a_dst = jnp.pad(a_dst, (0, nch * crow - nd))
    src = jnp.pad(e[0], (0, e_pad - ee))
    dst = jnp.pad(e[1], (0, e_pad - ee), constant_values=-1)

    edge_fn = _make_edge_pass(ns, nd, s, nch, own_n, eb, nst)
    acc = edge_fn(g_tab, a_dst, gmax.reshape(-1), src, dst)
    acc = acc[:nd]
    return _finish(acc, x_dst, p["W1"], p["b1"], p["W2"], p["b2"],
                   lnp=lnp, want_raw=want_raw)


def kernel(x_hit, x_nexus, x_interaction, x_ophit, x_pmt, x_flash, e_plane,
           e_hit_nexus, e_nexus_interaction, e_interaction_nexus, e_nexus_hit,
           e_ophit_pmt, e_pmt_flash, e_flash_interaction, e_interaction_flash,
           e_flash_pmt, e_pmt_ophit, params):
    P = params
    x_hit = _xattn(x_hit, x_flash, P["hit_flash_attention"])
    x_nexus = _xattn(x_nexus, x_pmt, P["nexus_pmt_attention"])
    x_interaction = _xattn(x_interaction, x_ophit,
                           P["interaction_ophit_attention"])

    (x_hit,) = _block_opt(x_hit, x_hit, e_plane, P["plane_net"])
    (x_nexus,) = _block_opt(x_hit, x_nexus, e_hit_nexus, P["plane_to_nexus"])
    (x_interaction,) = _block_opt(x_nexus, x_interaction, e_nexus_interaction,
                                  P["nexus_to_interaction"])
    x_nexus, x_nexus_ln = _block_opt(x_interaction, x_nexus,
                                     e_interaction_nexus,
                                     P["interaction_to_nexus"],
                                     lnp=P["nexus_norm"])
    (x_hit_ln,) = _block_opt(x_nexus, x_hit, e_nexus_hit, P["nexus_to_plane"],
                             lnp=P["hit_norm"], want_raw=False)
    (x_pmt,) = _block_opt(x_ophit, x_pmt, e_ophit_pmt, P["ophit_to_pmt"])
    (x_flash,) = _block_opt(x_pmt, x_flash, e_pmt_flash, P["pmt_to_flash"])
    x_interaction, x_interaction_ln = _block_opt(
        x_flash, x_interaction, e_flash_interaction, P["flash_to_interaction"],
        lnp=P["interaction_norm"])
    x_flash, x_flash_ln = _block_opt(x_interaction, x_flash,
                                     e_interaction_flash,
                                     P["interaction_to_flash"],
                                     lnp=P["flash_norm"])
    x_pmt, x_pmt_ln = _block_opt(x_flash, x_pmt, e_flash_pmt,
                                 P["flash_to_pmt"], lnp=P["pmt_norm"])
    (x_ophit_ln,) = _block_opt(x_pmt, x_ophit, e_pmt_ophit, P["pmt_to_ophit"],
                               lnp=P["ophit_norm"], want_raw=False)

    return (x_hit_ln, x_nexus_ln, x_interaction_ln, x_ophit_ln, x_pmt_ln,
            x_flash_ln)
